# baseline (speedup 1.0000x reference)
import functools

import jax
import jax.numpy as jnp
from jax.experimental import pallas as pl
from jax.experimental.pallas import tpu as pltpu

_B, _N, _D, _M = 2, 1024, 16, 16
_BI = 128


def _silu(x):
    return x * jax.nn.sigmoid(x)


def _egnn_block_kernel(
    featsT_ref,
    coorsT_ref,
    feats_i_ref,
    coors_i_ref,
    vel_i_ref,
    Wn1_ref,
    bn1_ref,
    Wn2_ref,
    bn2_ref,
    We1_s,
    be1_s,
    We2_s,
    be2_s,
    Wc1_s,
    bc1_s,
    Wc2_s,
    bc2_s,
    Wv_s,
    bv_s,
    h_ref,
    coors_o_ref,
    vel_o_ref,
):
    featsT = featsT_ref[0]
    coorsT = coorsT_ref[0]
    feats_i = feats_i_ref[0]
    coors_i = coors_i_ref[0]
    vel_i = vel_i_ref[0]

    dist = None
    for a in range(3):
        d = coors_i[:, a : a + 1] - coorsT[a : a + 1, :]
        dist = d * d if dist is None else dist + d * d

    ai = []
    bj = []
    for k in range(_M):
        acc_i = feats_i[:, 0:1] * We1_s[0, k]
        acc_j = featsT[0:1, :] * We1_s[_D, k]
        for d in range(1, _D):
            acc_i = acc_i + feats_i[:, d : d + 1] * We1_s[d, k]
            acc_j = acc_j + featsT[d : d + 1, :] * We1_s[_D + d, k]
        ai.append(acc_i)
        bj.append(acc_j)

    m1 = []
    for k in range(_M):
        x = ai[k] + bj[k] + dist * We1_s[2 * _D, k] + be1_s[0, k]
        m1.append(_silu(x))

    m2 = []
    for c in range(_M):
        acc = m1[0] * We2_s[0, c] + be2_s[0, c]
        for k in range(1, _M):
            acc = acc + m1[k] * We2_s[k, c]
        m2.append(_silu(acc))

    cw = None
    for c in range(_M):
        acc = m2[0] * Wc1_s[0, c] + bc1_s[0, c]
        for k in range(1, _M):
            acc = acc + m2[k] * Wc1_s[k, c]
        p = _silu(acc) * Wc2_s[c, 0]
        cw = p if cw is None else cw + p
    cw = cw + bc2_s[0, 0]

    m_i = jnp.concatenate(
        [jnp.sum(m2[k], axis=1, keepdims=True) for k in range(_M)], axis=1
    )
    sum_cw = jnp.sum(cw, axis=1, keepdims=True)
    cwx = jnp.concatenate(
        [
            jnp.sum(cw * coorsT[a : a + 1, :], axis=1, keepdims=True)
            for a in range(3)
        ],
        axis=1,
    )
    agg = (sum_cw * coors_i - cwx) * (1.0 / _N)

    gate = feats_i[:, 0:1] * Wv_s[0, 0]
    for d in range(1, _D):
        gate = gate + feats_i[:, d : d + 1] * Wv_s[d, 0]
    gate = gate + bv_s[0, 0]
    vel_new = gate * vel_i + agg
    coors_new = coors_i + vel_new

    node_in = jnp.concatenate([feats_i, m_i], axis=1)
    h1 = _silu(
        jax.lax.dot_general(
            node_in,
            Wn1_ref[...],
            (((1,), (0,)), ((), ())),
            preferred_element_type=jnp.float32,
            precision=jax.lax.Precision.HIGHEST,
        )
        + bn1_ref[...]
    )
    h2 = (
        jax.lax.dot_general(
            h1,
            Wn2_ref[...],
            (((1,), (0,)), ((), ())),
            preferred_element_type=jnp.float32,
            precision=jax.lax.Precision.HIGHEST,
        )
        + bn2_ref[...]
    )
    h_new = feats_i + h2

    h_ref[0] = h_new
    coors_o_ref[0] = coors_new
    vel_o_ref[0] = vel_new


@jax.jit
def kernel(feats, coors, vel, We1, be1, We2, be2, Wc1, bc1, Wc2, bc2,
           Wv, bv, Wn1, bn1, Wn2, bn2):
    featsT = jnp.transpose(feats, (0, 2, 1))
    coorsT = jnp.transpose(coors, (0, 2, 1))

    row = lambda x: jnp.reshape(x, (1, -1))

    grid = (_B, _N // _BI)

    full2d = lambda shape: pl.BlockSpec(shape, lambda b, i: (0, 0))
    smem = lambda shape: pl.BlockSpec(
        shape, lambda b, i: (0, 0), memory_space=pltpu.SMEM
    )

    out_shapes = (
        jax.ShapeDtypeStruct((_B, _N, _D), jnp.float32),
        jax.ShapeDtypeStruct((_B, _N, 3), jnp.float32),
        jax.ShapeDtypeStruct((_B, _N, 3), jnp.float32),
    )

    h_new, coors_new, vel_new = pl.pallas_call(
        _egnn_block_kernel,
        grid=grid,
        in_specs=[
            pl.BlockSpec((1, _D, _N), lambda b, i: (b, 0, 0)),
            pl.BlockSpec((1, 3, _N), lambda b, i: (b, 0, 0)),
            pl.BlockSpec((1, _BI, _D), lambda b, i: (b, i, 0)),
            pl.BlockSpec((1, _BI, 3), lambda b, i: (b, i, 0)),
            pl.BlockSpec((1, _BI, 3), lambda b, i: (b, i, 0)),
            full2d((2 * _D, _D)),
            full2d((1, _D)),
            full2d((_D, _D)),
            full2d((1, _D)),
            smem((2 * _D + 1, _M)),
            smem((1, _M)),
            smem((_M, _M)),
            smem((1, _M)),
            smem((_M, _M)),
            smem((1, _M)),
            smem((_M, 1)),
            smem((1, 1)),
            smem((_D, 1)),
            smem((1, 1)),
        ],
        out_specs=[
            pl.BlockSpec((1, _BI, _D), lambda b, i: (b, i, 0)),
            pl.BlockSpec((1, _BI, 3), lambda b, i: (b, i, 0)),
            pl.BlockSpec((1, _BI, 3), lambda b, i: (b, i, 0)),
        ],
        out_shape=out_shapes,
    )(
        featsT, coorsT, feats, coors, vel,
        Wn1, row(bn1), Wn2, row(bn2),
        We1, row(be1), We2, row(be2), Wc1, row(bc1), Wc2, row(bc2),
        jnp.reshape(Wv, (_D, 1)), row(bv),
    )
    return (h_new, coors_new, vel_new)


# speedup vs baseline: 2.6995x; 2.6995x over previous
import jax
import jax.numpy as jnp
from jax.experimental import pallas as pl
from jax.experimental.pallas import tpu as pltpu

_B, _N, _D, _M = 2, 1024, 16, 16
_BI = 128
_G = 16
_NG = _BI // _G


def _silu(x):
    return x * jax.nn.sigmoid(x)


def _egnn_kernel(
    featsbf_ref,
    coorsA_ref,
    coorsTi_ref,
    featsP_ref,
    coorsP_ref,
    velP_ref,
    W1t_ref,
    W2bd_ref,
    W3bd_ref,
    W4bd_ref,
    b2row_ref,
    b3row_ref,
    WaiK_ref,
    WvK_ref,
    Wn1aK_ref,
    Wn1bK_ref,
    Wn2K_ref,
    bn1row_ref,
    bn2row_ref,
    bc2_s,
    bv_s,
    hP_ref,
    coorsPn_ref,
    velPn_ref,
):
    feats_bf = featsbf_ref[0]
    coors_all = coorsA_ref[0]
    xi = coorsTi_ref[0]
    P = featsP_ref[0]
    coorsP = coorsP_ref[0]
    velP = velP_ref[0]

    f32 = jnp.float32
    bf16 = jnp.bfloat16
    hi = jax.lax.Precision.HIGHEST

    def mm(a, b, prec=None):
        return jax.lax.dot_general(
            a, b, (((1,), (0,)), ((), ())),
            preferred_element_type=f32, precision=prec)

    dist = None
    for a in range(3):
        d = coors_all[:, a : a + 1] - xi[a : a + 1, :]
        dist = d * d if dist is None else dist + d * d
    dist_h = dist.astype(bf16)
    dist_l = (dist - dist_h.astype(f32)).astype(bf16)

    ai_rows = mm(P, WaiK_ref[...], hi)

    ones_col = jnp.ones((_N, 1), dtype=bf16)
    W1t = W1t_ref[...]
    W2bd = W2bd_ref[...]
    W3bd = W3bd_ref[...]
    W4bd = W4bd_ref[...]
    b2row = b2row_ref[...]
    b3row = b3row_ref[...]
    bc2 = bc2_s[0, 0]

    msum_rows = []
    cwsum_rows = []
    cwx_rows = [[], [], []]
    for g in range(_NG):
        sl = slice(_G * g, _G * (g + 1))
        X = jnp.concatenate(
            [feats_bf, dist_h[:, sl], dist_l[:, sl], ones_col], axis=1
        )
        z1 = mm(X, W1t) + ai_rows[g : g + 1, :]
        m1 = _silu(z1)
        z2 = mm(m1.astype(bf16), W2bd) + b2row
        m2 = _silu(z2)
        msum_rows.append(jnp.sum(m2, axis=0, keepdims=True))
        z3 = mm(m2.astype(bf16), W3bd) + b3row
        p3 = _silu(z3)
        cw = mm(p3.astype(bf16), W4bd) + bc2
        cwsum_rows.append(jnp.sum(cw, axis=0, keepdims=True))
        for a in range(3):
            cwx_rows[a].append(
                jnp.sum(cw * coors_all[:, a : a + 1], axis=0, keepdims=True)
            )

    Msum = jnp.concatenate(msum_rows, axis=0)
    SC = jnp.concatenate(cwsum_rows, axis=0)

    gate = mm(P, WvK_ref[...], hi) + bv_s[0, 0]

    inv_n = 1.0 / _N
    for a in range(3):
        CXa = jnp.concatenate(cwx_rows[a], axis=0)
        agg_a = (SC * coorsP[a] - CXa) * inv_n
        vel_a = gate * velP[a] + agg_a
        velPn_ref[0, a] = vel_a
        coorsPn_ref[0, a] = coorsP[a] + vel_a

    n1 = _silu(
        mm(P, Wn1aK_ref[...], hi)
        + mm(Msum, Wn1bK_ref[...], hi)
        + bn1row_ref[...]
    )
    h2 = mm(n1, Wn2K_ref[...], hi) + bn2row_ref[...]
    hP_ref[0] = P + h2


@jax.jit
def kernel(feats, coors, vel, We1, be1, We2, be2, Wc1, bc1, Wc2, bc2,
           Wv, bv, Wn1, bn1, Wn2, bn2):
    f32 = jnp.float32
    bf16 = jnp.bfloat16
    eye = jnp.eye(_G, dtype=f32)

    feats_bf = feats.astype(bf16)
    coorsT = jnp.transpose(coors, (0, 2, 1))
    featsP = jnp.reshape(feats, (_B, _N // _G, _G * _D))
    coorsP = jnp.reshape(coorsT, (_B, 3, _N // _G, _G))
    velP = jnp.reshape(jnp.transpose(vel, (0, 2, 1)), (_B, 3, _N // _G, _G))

    wd = We1[2 * _D, :]
    W1t = jnp.concatenate(
        [
            jnp.tile(We1[_D : 2 * _D, :], (1, _G)),
            jnp.kron(eye, wd[None, :]),
            jnp.kron(eye, wd[None, :]),
            jnp.tile(be1[None, :], (1, _G)),
        ],
        axis=0,
    ).astype(bf16)
    W2bd = jnp.kron(eye, We2).astype(bf16)
    W3bd = jnp.kron(eye, Wc1).astype(bf16)
    W4bd = jnp.kron(eye, Wc2).astype(bf16)
    b2row = jnp.tile(be2[None, :], (1, _G))
    b3row = jnp.tile(bc1[None, :], (1, _G))
    WaiK = jnp.kron(eye, We1[:_D, :])
    WvK = jnp.kron(eye, jnp.reshape(Wv, (_D, 1)))
    Wn1aK = jnp.kron(eye, Wn1[:_D, :])
    Wn1bK = jnp.kron(eye, Wn1[_D:, :])
    Wn2K = jnp.kron(eye, Wn2)
    bn1row = jnp.tile(bn1[None, :], (1, _G))
    bn2row = jnp.tile(bn2[None, :], (1, _G))

    grid = (_B, _N // _BI)
    ng = _BI // _G

    full = lambda shape: pl.BlockSpec(shape, lambda b, i: tuple(0 for _ in shape))
    smem = lambda shape: pl.BlockSpec(
        shape, lambda b, i: tuple(0 for _ in shape), memory_space=pltpu.SMEM
    )

    out_shapes = (
        jax.ShapeDtypeStruct((_B, _N // _G, _G * _D), f32),
        jax.ShapeDtypeStruct((_B, 3, _N // _G, _G), f32),
        jax.ShapeDtypeStruct((_B, 3, _N // _G, _G), f32),
    )

    hP, coorsPn, velPn = pl.pallas_call(
        _egnn_kernel,
        grid=grid,
        in_specs=[
            pl.BlockSpec((1, _N, _D), lambda b, i: (b, 0, 0)),
            pl.BlockSpec((1, _N, 3), lambda b, i: (b, 0, 0)),
            pl.BlockSpec((1, 3, _BI), lambda b, i: (b, 0, i)),
            pl.BlockSpec((1, ng, _G * _D), lambda b, i: (b, i, 0)),
            pl.BlockSpec((1, 3, ng, _G), lambda b, i: (b, 0, i, 0)),
            pl.BlockSpec((1, 3, ng, _G), lambda b, i: (b, 0, i, 0)),
            full((2 * _D + 17, _G * _M)),
            full((_G * _M, _G * _M)),
            full((_G * _M, _G * _M)),
            full((_G * _M, _G)),
            full((1, _G * _M)),
            full((1, _G * _M)),
            full((_G * _D, _G * _M)),
            full((_G * _D, _G)),
            full((_G * _D, _G * _D)),
            full((_G * _M, _G * _D)),
            full((_G * _D, _G * _D)),
            full((1, _G * _D)),
            full((1, _G * _D)),
            smem((1, 1)),
            smem((1, 1)),
        ],
        out_specs=[
            pl.BlockSpec((1, ng, _G * _D), lambda b, i: (b, i, 0)),
            pl.BlockSpec((1, 3, ng, _G), lambda b, i: (b, 0, i, 0)),
            pl.BlockSpec((1, 3, ng, _G), lambda b, i: (b, 0, i, 0)),
        ],
        out_shape=out_shapes,
    )(
        feats_bf, coors, coorsT, featsP, coorsP, velP,
        W1t, W2bd, W3bd, W4bd, b2row, b3row,
        WaiK, WvK, Wn1aK, Wn1bK, Wn2K, bn1row, bn2row,
        jnp.reshape(bc2, (1, 1)), jnp.reshape(bv, (1, 1)),
    )

    h_new = jnp.reshape(hP, (_B, _N, _D))
    coors_new = jnp.transpose(jnp.reshape(coorsPn, (_B, 3, _N)), (0, 2, 1))
    vel_new = jnp.transpose(jnp.reshape(velPn, (_B, 3, _N)), (0, 2, 1))
    return (h_new, coors_new, vel_new)
